# baseline (speedup 1.0000x reference)
import dataclasses
import functools

import jax
import jax.numpy as jnp
from jax import lax
from jax.experimental import pallas as pl
from jax.experimental.pallas import tpu as pltpu
from jax.experimental.pallas import tpu_sc as plsc

_PAD = 1
_EPS = 1e-05
_HIDDEN = 768

_NC = 2
_NS = 16
_NW = _NC * _NS
_CH = 16
_NB = 4


def _posid_body(ids_ref, oid_ref):
    ids = ids_ref[...]
    mask = (ids != _PAD).astype(jnp.int32)
    x = mask
    seq = ids.shape[1]
    k = 1
    while k < seq:
        shifted = jnp.concatenate(
            [jnp.zeros((ids.shape[0], k), jnp.int32), x[:, :-k]], axis=1)
        x = x + shifted
        k *= 2
    oid_ref[...] = x * mask + _PAD


def _position_ids(input_ids):
    return pl.pallas_call(
        _posid_body,
        out_shape=jax.ShapeDtypeStruct(input_ids.shape, jnp.int32),
    )(input_ids)


@functools.lru_cache(maxsize=None)
def _make_gather_add(v_word, v_pos, d, nrow, seq):
    b = nrow * seq
    rpw = b // _NW
    nch = rpw // _CH
    wps = seq // rpw
    assert nch % _NB == 0 and nch >= 2 * _NB and wps * rpw == seq
    mesh = plsc.VectorSubcoreMesh(core_axis_name="c", subcore_axis_name="s")
    cp = pltpu.CompilerParams()
    if "needs_layout_passes" in pltpu.CompilerParams.__dataclass_fields__:
        cp = dataclasses.replace(cp, needs_layout_passes=False)

    @functools.partial(
        pl.kernel,
        mesh=mesh,
        compiler_params=cp,
        out_type=jax.ShapeDtypeStruct((b, d // 2), jnp.int32),
        scratch_types=[
            pltpu.VMEM((rpw,), jnp.int32),
            pltpu.VMEM((rpw,), jnp.int32),
            pltpu.VMEM((_NB, _CH, d), jnp.float32),
            pltpu.VMEM((_NB, _CH, d), jnp.float32),
            pltpu.VMEM((_NB, _CH, d // 2), jnp.int32),
        ] + [pltpu.SemaphoreType.DMA] * (3 * _NB),
    )
    def gather_kernel(word_hbm, pos_hbm, iw_hbm, ip_hbm, out_hbm,
                      iw_v, ip_v, wbuf, pbuf, obuf, *sems):
        semw = sems[0:_NB]
        semp = sems[_NB:2 * _NB]
        semo = sems[2 * _NB:3 * _NB]
        wid = lax.axis_index("s") * _NC + lax.axis_index("c")
        base = wid * rpw
        srow = wid // wps
        scol = (wid % wps) * rpw

        def fire(cc, bb):
            pltpu.async_copy(
                word_hbm.at[iw_v.at[pl.ds(cc * _CH, _CH)]], wbuf.at[bb],
                semw[bb])
            pltpu.async_copy(
                pos_hbm.at[ip_v.at[pl.ds(cc * _CH, _CH)]], pbuf.at[bb],
                semp[bb])

        def wait_gather(bb):
            pltpu.make_async_copy(
                word_hbm.at[pl.ds(0, _CH)], wbuf.at[bb], semw[bb]).wait()
            pltpu.make_async_copy(
                pos_hbm.at[pl.ds(0, _CH)], pbuf.at[bb], semp[bb]).wait()

        def wait_out(bb):
            pltpu.make_async_copy(
                obuf.at[bb], out_hbm.at[pl.ds(base, _CH)], semo[bb]).wait()

        pltpu.sync_copy(iw_hbm.at[srow, pl.ds(scol, rpw)], iw_v)
        pltpu.sync_copy(ip_hbm.at[srow, pl.ds(scol, rpw)], ip_v)
        fire(0, 0)
        fire(1, 1)

        @pl.loop(0, nch, step=_NB)
        def _(c):
            for bb in range(_NB):
                cc = c + bb
                fb = (bb + 2) % _NB

                @pl.when(cc + 2 < nch)
                def _():
                    @pl.when(cc >= 2)
                    def _():
                        wait_out(fb)

                    fire(cc + 2, fb)

                wait_gather(bb)

                @plsc.parallel_loop(0, _CH, step=1, unroll=2)
                def _(r):
                    h = d // 2
                    for col in range(0, h, 16):
                        a = (wbuf[bb, r, pl.ds(col, 16)]
                             + pbuf[bb, r, pl.ds(col, 16)])
                        z = (wbuf[bb, r, pl.ds(col + h, 16)]
                             + pbuf[bb, r, pl.ds(col + h, 16)])
                        ai = plsc.bitcast(a, jnp.int32) + 0x8000
                        zi = plsc.bitcast(z, jnp.int32) + 0x8000
                        obuf[bb, r, pl.ds(col, 16)] = (
                            lax.shift_right_logical(ai, 16)
                            | (zi & jnp.int32(-65536)))

                pltpu.async_copy(
                    obuf.at[bb], out_hbm.at[pl.ds(base + cc * _CH, _CH)],
                    semo[bb])

        for bb in range(_NB):
            wait_out(bb)

    return gather_kernel


def _ln_body(s_ref, t_ref, lw_ref, lb_ref, o_ref):
    p = s_ref[...]
    lo = lax.bitcast_convert_type(p << 16, jnp.float32)
    hi = lax.bitcast_convert_type(p & jnp.int32(-65536), jnp.float32)
    x = jnp.concatenate([lo, hi], axis=1) + t_ref[0:1, :]
    mean = jnp.mean(x, axis=-1, keepdims=True)
    m2 = jnp.mean(x * x, axis=-1, keepdims=True)
    var = m2 - mean * mean
    o_ref[...] = (x - mean) * lax.rsqrt(var + _EPS) * lw_ref[...] + lb_ref[...]


def _ln(sum_rows, type_emb, ln_w, ln_b):
    b, dh = sum_rows.shape
    d = dh * 2
    rb = 2048
    grid = (b // rb,)
    return pl.pallas_call(
        _ln_body,
        grid=grid,
        in_specs=[
            pl.BlockSpec((rb, dh), lambda i: (i, 0)),
            pl.BlockSpec(type_emb.shape, lambda i: (0, 0)),
            pl.BlockSpec((1, d), lambda i: (0, 0)),
            pl.BlockSpec((1, d), lambda i: (0, 0)),
        ],
        out_specs=pl.BlockSpec((rb, d), lambda i: (i, 0)),
        out_shape=jax.ShapeDtypeStruct((b, d), jnp.float32),
    )(sum_rows, type_emb, ln_w, ln_b)


def kernel(input_ids, word_emb, pos_emb, type_emb, ln_w, ln_b):
    bb, seq = input_ids.shape
    d = word_emb.shape[1]
    b = bb * seq

    position_ids = _position_ids(input_ids)

    gather = _make_gather_add(word_emb.shape[0], pos_emb.shape[0], d, bb, seq)
    sum_rows = gather(word_emb, pos_emb, input_ids, position_ids)

    out = _ln(sum_rows, type_emb,
              ln_w.reshape(1, d), ln_b.reshape(1, d))
    return out.reshape(bb, seq, d)


# speedup vs baseline: 1.1050x; 1.1050x over previous
import dataclasses
import functools

import jax
import jax.numpy as jnp
from jax import lax
from jax.experimental import pallas as pl
from jax.experimental.pallas import tpu as pltpu
from jax.experimental.pallas import tpu_sc as plsc

_PAD = 1
_EPS = 1e-05
_HIDDEN = 768

_NC = 2
_NS = 16
_NW = _NC * _NS
_CH = 16
_NB = 4


def _posid_body(ids_ref, oid_ref):
    ids = ids_ref[...]
    mask = (ids != _PAD).astype(jnp.int32)
    x = mask
    seq = ids.shape[1]
    k = 1
    while k < seq:
        shifted = jnp.concatenate(
            [jnp.zeros((ids.shape[0], k), jnp.int32), x[:, :-k]], axis=1)
        x = x + shifted
        k *= 2
    oid_ref[...] = x * mask + _PAD


def _position_ids(input_ids):
    return pl.pallas_call(
        _posid_body,
        out_shape=jax.ShapeDtypeStruct(input_ids.shape, jnp.int32),
    )(input_ids)


@functools.lru_cache(maxsize=None)
def _make_gather_add(v_word, v_pos, d, nrow, seq):
    b = nrow * seq
    rpw = b // _NW
    nch = rpw // _CH
    wps = seq // rpw
    assert nch % _NB == 0 and nch >= 2 * _NB and wps * rpw == seq
    mesh = plsc.VectorSubcoreMesh(core_axis_name="c", subcore_axis_name="s")
    cp = pltpu.CompilerParams()
    if "needs_layout_passes" in pltpu.CompilerParams.__dataclass_fields__:
        cp = dataclasses.replace(cp, needs_layout_passes=False)

    @functools.partial(
        pl.kernel,
        mesh=mesh,
        compiler_params=cp,
        out_type=jax.ShapeDtypeStruct((b, d), jnp.float32),
        scratch_types=[
            pltpu.VMEM((rpw,), jnp.int32),
            pltpu.VMEM((rpw,), jnp.int32),
            pltpu.VMEM((_NB, _CH, d), jnp.float32),
            pltpu.VMEM((_NB, _CH, d), jnp.float32),
        ] + [pltpu.SemaphoreType.DMA] * (3 * _NB),
    )
    def gather_kernel(word_hbm, pos_hbm, iw_hbm, ip_hbm, out_hbm,
                      iw_v, ip_v, wbuf, pbuf, *sems):
        semw = sems[0:_NB]
        semp = sems[_NB:2 * _NB]
        semo = sems[2 * _NB:3 * _NB]
        wid = lax.axis_index("s") * _NC + lax.axis_index("c")
        base = wid * rpw
        srow = wid // wps
        scol = (wid % wps) * rpw

        def fire(cc, bb):
            pltpu.async_copy(
                word_hbm.at[iw_v.at[pl.ds(cc * _CH, _CH)]], wbuf.at[bb],
                semw[bb])
            pltpu.async_copy(
                pos_hbm.at[ip_v.at[pl.ds(cc * _CH, _CH)]], pbuf.at[bb],
                semp[bb])

        def wait_gather(bb):
            pltpu.make_async_copy(
                word_hbm.at[pl.ds(0, _CH)], wbuf.at[bb], semw[bb]).wait()
            pltpu.make_async_copy(
                pos_hbm.at[pl.ds(0, _CH)], pbuf.at[bb], semp[bb]).wait()

        def wait_out(bb):
            pltpu.make_async_copy(
                wbuf.at[bb], out_hbm.at[pl.ds(base, _CH)], semo[bb]).wait()

        pltpu.sync_copy(iw_hbm.at[srow, pl.ds(scol, rpw)], iw_v)
        pltpu.sync_copy(ip_hbm.at[srow, pl.ds(scol, rpw)], ip_v)
        fire(0, 0)
        fire(1, 1)

        @pl.loop(0, nch, step=_NB)
        def _(c):
            for bb in range(_NB):
                cc = c + bb
                fb = (bb + 2) % _NB

                @pl.when(cc + 2 < nch)
                def _():
                    @pl.when(cc >= 2)
                    def _():
                        wait_out(fb)

                    fire(cc + 2, fb)

                wait_gather(bb)

                @plsc.parallel_loop(0, _CH, step=1, unroll=2)
                def _(r):
                    for col in range(0, d, 16):
                        wbuf[bb, r, pl.ds(col, 16)] = (
                            wbuf[bb, r, pl.ds(col, 16)]
                            + pbuf[bb, r, pl.ds(col, 16)])

                pltpu.async_copy(
                    wbuf.at[bb], out_hbm.at[pl.ds(base + cc * _CH, _CH)],
                    semo[bb])

        for bb in range(_NB):
            wait_out(bb)

    return gather_kernel


def _ln_body(s_ref, t_ref, lw_ref, lb_ref, o_ref):
    x = s_ref[...] + t_ref[0:1, :]
    mean = jnp.mean(x, axis=-1, keepdims=True)
    m2 = jnp.mean(x * x, axis=-1, keepdims=True)
    var = m2 - mean * mean
    o_ref[...] = (x - mean) * lax.rsqrt(var + _EPS) * lw_ref[...] + lb_ref[...]


def _ln(sum_rows, type_emb, ln_w, ln_b):
    b, d = sum_rows.shape
    rb = 2048
    grid = (b // rb,)
    return pl.pallas_call(
        _ln_body,
        grid=grid,
        in_specs=[
            pl.BlockSpec((rb, d), lambda i: (i, 0)),
            pl.BlockSpec(type_emb.shape, lambda i: (0, 0)),
            pl.BlockSpec((1, d), lambda i: (0, 0)),
            pl.BlockSpec((1, d), lambda i: (0, 0)),
        ],
        out_specs=pl.BlockSpec((rb, d), lambda i: (i, 0)),
        out_shape=jax.ShapeDtypeStruct((b, d), jnp.float32),
    )(sum_rows, type_emb, ln_w, ln_b)


def kernel(input_ids, word_emb, pos_emb, type_emb, ln_w, ln_b):
    bb, seq = input_ids.shape
    d = word_emb.shape[1]
    b = bb * seq

    position_ids = _position_ids(input_ids)

    gather = _make_gather_add(word_emb.shape[0], pos_emb.shape[0], d, bb, seq)
    sum_rows = gather(word_emb, pos_emb, input_ids, position_ids)

    out = _ln(sum_rows, type_emb,
              ln_w.reshape(1, d), ln_b.reshape(1, d))
    return out.reshape(bb, seq, d)
